# baseline (speedup 1.0000x reference)
import functools

import jax
import jax.numpy as jnp
from jax.experimental import pallas as pl
from jax.experimental.pallas import tpu as pltpu


def _fused_net_kernel(x_ref, w1_ref, t1_ref, w2_ref, s2_ref, t2_ref,
                      fw1_ref, s3_ref, t3_ref, fw2_ref, fb2_ref, o_ref, *, bt):
    p = x_ref[...]
    h = jnp.dot(p, w1_ref[0], preferred_element_type=jnp.float32)
    for m in range(1, 9):
        h = jnp.maximum(
            h, jnp.dot(p, w1_ref[m], preferred_element_type=jnp.float32))
    h = jnp.maximum(h + t1_ref[...], 0.0)
    y1 = h.astype(jnp.bfloat16)

    acc = jnp.dot(y1, w2_ref[...], preferred_element_type=jnp.float32)
    z = jnp.maximum(acc * s2_ref[...] + t2_ref[...], 0.0)
    z = jnp.concatenate(
        [jnp.maximum(z[:, 0:200], z[:, 200:400]),
         jnp.maximum(z[:, 400:600], z[:, 600:800])], axis=1)
    flat = jnp.concatenate(
        [jnp.maximum(z[:, o:o + 50], z[:, o + 50:o + 100])
         for o in (0, 100, 200, 300)], axis=1)
    flat = flat.astype(jnp.bfloat16)

    g = jnp.dot(flat, fw1_ref[...], preferred_element_type=jnp.float32)
    g = jnp.maximum(g * s3_ref[...] + t3_ref[...], 0.0).astype(jnp.bfloat16)
    o = jnp.dot(g, fw2_ref[...], preferred_element_type=jnp.float32)
    o_ref[...] = o + fb2_ref[...]


def _band_conv1(w1r, s1):
    w = w1r.reshape(5, 5, 20) * s1.reshape(1, 1, 20)
    hh = jnp.arange(28)[:, None]
    oo = jnp.arange(8)[None, :]
    mats = []
    for i in range(3):
        dh = hh - (3 * oo + i)
        vh = (dh >= 0) & (dh < 5)
        for j in range(3):
            dj = hh - (3 * oo + j)
            vj = (dj >= 0) & (dj < 5)
            t = w[jnp.clip(dh, 0, 4)[:, None, :, None],
                  jnp.clip(dj, 0, 4)[None, :, None, :], :]
            t = jnp.where(vh[:, None, :, None, None]
                          & vj[None, :, None, :, None], t, 0.0)
            mats.append(t.reshape(784, 1280))
    return jnp.stack(mats)


def _band_conv2(w2r):
    w = w2r.reshape(5, 5, 20, 50)
    hy = jnp.arange(8)[:, None]
    ho = jnp.arange(4)[None, :]
    d = hy - ho
    v = (d >= 0) & (d < 5)
    dc = jnp.clip(d, 0, 4)
    t = w[dc[:, None, :, None], dc[None, :, None, :], :, :]
    t = jnp.where(v[:, None, :, None, None, None]
                  & v[None, :, None, :, None, None], t, jnp.bfloat16(0))
    return t.transpose(0, 1, 4, 2, 3, 5).reshape(1280, 800)


def kernel(x, w1r, s1, t1, w2r, s2, t2, fw1p, s3, t3, fw2t, fb2r):
    N = x.shape[0]
    bt = 256
    xf = x.reshape(N, 784)
    w1b = _band_conv1(w1r, s1)
    t1t = jnp.tile(t1, (1, 64))
    w2b = _band_conv2(w2r)
    s2t = jnp.tile(s2, (1, 16))
    t2t = jnp.tile(t2, (1, 16))

    fn = functools.partial(_fused_net_kernel, bt=bt)
    return pl.pallas_call(
        fn,
        out_shape=jax.ShapeDtypeStruct((N, 10), jnp.float32),
        grid=(N // bt,),
        in_specs=[
            pl.BlockSpec((bt, 784), lambda n: (n, 0)),
            pl.BlockSpec((9, 784, 1280), lambda n: (0, 0, 0)),
            pl.BlockSpec((1, 1280), lambda n: (0, 0)),
            pl.BlockSpec((1280, 800), lambda n: (0, 0)),
            pl.BlockSpec((1, 800), lambda n: (0, 0)),
            pl.BlockSpec((1, 800), lambda n: (0, 0)),
            pl.BlockSpec((200, 500), lambda n: (0, 0)),
            pl.BlockSpec((1, 500), lambda n: (0, 0)),
            pl.BlockSpec((1, 500), lambda n: (0, 0)),
            pl.BlockSpec((500, 10), lambda n: (0, 0)),
            pl.BlockSpec((1, 10), lambda n: (0, 0)),
        ],
        out_specs=pl.BlockSpec((bt, 10), lambda n: (n, 0)),
        compiler_params=pltpu.CompilerParams(
            dimension_semantics=("parallel",)),
    )(xf, w1b, t1t, w2b, s2t, t2t, fw1p, s3, t3, fw2t, fb2r)


# speedup vs baseline: 19.0220x; 19.0220x over previous
import functools

import jax
import jax.numpy as jnp
from jax.experimental import pallas as pl
from jax.experimental.pallas import tpu as pltpu


def _fused_net_kernel(x_ref, w1_ref, t1_ref, w2_ref, s2_ref, t2_ref,
                      fw1_ref, s3_ref, t3_ref, fw2_ref, fb2_ref, o_ref, *, bt):
    x = x_ref[...]
    pieces = []
    for r in range(7):
        a, b = divmod(r, 3)
        pieces.append(x[:, b, a:a + 8, :].reshape(bt * 8, 28))
    p = jnp.concatenate(pieces, axis=1)
    h = jnp.dot(p, w1_ref[0], preferred_element_type=jnp.float32)
    for m in range(1, 9):
        h = jnp.maximum(
            h, jnp.dot(p, w1_ref[m], preferred_element_type=jnp.float32))
    h = jnp.maximum(h + t1_ref[...], 0.0)
    y1 = h.astype(jnp.bfloat16).reshape(bt, 8, 160)

    acc = jnp.zeros((bt * 4, 200), jnp.float32)
    for ki in range(5):
        q = y1[:, ki:ki + 4, :].reshape(bt * 4, 160)
        acc = acc + jnp.dot(q, w2_ref[ki], preferred_element_type=jnp.float32)
    z = jnp.maximum(acc * s2_ref[...] + t2_ref[...], 0.0)
    z = jnp.concatenate(
        [jnp.maximum(z[:, 0:50], z[:, 50:100]),
         jnp.maximum(z[:, 100:150], z[:, 150:200])], axis=1)
    z = jnp.max(z.reshape(bt * 2, 2, 100), axis=1)
    z = z.reshape(bt, 2, 100)
    flat = jnp.concatenate([z[:, 0, :], z[:, 1, :]], axis=1)
    flat = flat.astype(jnp.bfloat16)

    g = jnp.dot(flat, fw1_ref[...], preferred_element_type=jnp.float32)
    g = jnp.maximum(g * s3_ref[...] + t3_ref[...], 0.0).astype(jnp.bfloat16)
    o = jnp.dot(g, fw2_ref[...], preferred_element_type=jnp.float32)
    o_ref[...] = o + fb2_ref[...]


def _band_conv1(w1r, s1):
    w = w1r.reshape(5, 5, 20) * s1.reshape(1, 1, 20)
    ki = jnp.arange(5)
    i3 = jnp.arange(3)
    a = (jnp.arange(7)[None, None, :]
         == ki[None, :, None] + i3[:, None, None]).astype(jnp.float32)
    b = (jnp.arange(28)[None, None, :, None]
         == (ki[None, :, None, None] + 3 * jnp.arange(8)[None, None, None, :]
             + i3[:, None, None, None])).astype(jnp.float32)
    t = jnp.einsum('ikr,jlwz,klc->ijrwzc', a, b, w)
    return t.reshape(9, 196, 160)


def _band_conv2(w2r):
    w = w2r.reshape(5, 5, 20, 50).astype(jnp.float32)
    c = (jnp.arange(8)[None, :, None]
         == jnp.arange(5)[:, None, None]
         + jnp.arange(4)[None, None, :]).astype(jnp.float32)
    t = jnp.einsum('lwo,klcd->kwcod', c, w)
    return t.reshape(5, 160, 200).astype(jnp.bfloat16)


def kernel(x, w1r, s1, t1, w2r, s2, t2, fw1p, s3, t3, fw2t, fb2r):
    N = x.shape[0]
    bt = 256
    xg = jnp.pad(x.reshape(N, 28, 28), ((0, 0), (0, 2), (0, 0)))
    xg = xg.reshape(N, 10, 3, 28).transpose(0, 2, 1, 3)
    w1b = _band_conv1(w1r, s1)
    t1t = jnp.tile(t1, (1, 8))
    w2b = _band_conv2(w2r)
    s2t = jnp.tile(s2, (1, 4))
    t2t = jnp.tile(t2, (1, 4))

    fn = functools.partial(_fused_net_kernel, bt=bt)
    return pl.pallas_call(
        fn,
        out_shape=jax.ShapeDtypeStruct((N, 10), jnp.float32),
        grid=(N // bt,),
        in_specs=[
            pl.BlockSpec((bt, 3, 10, 28), lambda n: (n, 0, 0, 0)),
            pl.BlockSpec((9, 196, 160), lambda n: (0, 0, 0)),
            pl.BlockSpec((1, 160), lambda n: (0, 0)),
            pl.BlockSpec((5, 160, 200), lambda n: (0, 0, 0)),
            pl.BlockSpec((1, 200), lambda n: (0, 0)),
            pl.BlockSpec((1, 200), lambda n: (0, 0)),
            pl.BlockSpec((200, 500), lambda n: (0, 0)),
            pl.BlockSpec((1, 500), lambda n: (0, 0)),
            pl.BlockSpec((1, 500), lambda n: (0, 0)),
            pl.BlockSpec((500, 10), lambda n: (0, 0)),
            pl.BlockSpec((1, 10), lambda n: (0, 0)),
        ],
        out_specs=pl.BlockSpec((bt, 10), lambda n: (n, 0)),
        compiler_params=pltpu.CompilerParams(
            dimension_semantics=("parallel",)),
    )(xg, w1b, t1t, w2b, s2t, t2t, fw1p, s3, t3, fw2t, fb2r)
